# baseline (speedup 1.0000x reference)
import functools

import jax
import jax.numpy as jnp
from jax import lax
from jax.experimental import pallas as pl
from jax.experimental.pallas import tpu as pltpu
from jax.experimental.pallas import tpu_sc as plsc

CLS = 100000
FEAT = 64
BATCH_N = 16384

_NC = 2
_NS = 16
_NW = _NC * _NS
_FPW = FEAT // _NW // 2
_L = 16
_CHUNK = BATCH_N // 4
_GRP = 4
_ITERS = _CHUNK // (_L * _GRP)


def _make_center_partials(half):
  @functools.partial(
      pl.kernel,
      out_type=jax.ShapeDtypeStruct((_NW * _L,), jnp.float32),
      mesh=plsc.VectorSubcoreMesh(
          core_axis_name="c", subcore_axis_name="s",
          num_cores=_NC, num_subcores=_NS,
      ),
      scratch_types=[
          pltpu.VMEM((CLS,), jnp.float32),
          pltpu.VMEM((BATCH_N,), jnp.int32),
          pltpu.VMEM((2, _CHUNK), jnp.float32),
          pltpu.VMEM((_L,), jnp.float32),
          pltpu.SemaphoreType.DMA,
          pltpu.SemaphoreType.DMA,
          pltpu.SemaphoreType.DMA,
      ],
      compiler_params=pltpu.CompilerParams(needs_layout_passes=False),
      name=f"center_partials_{half}",
  )
  def _center_partials(xs_t_hbm, label_hbm, center_t_hbm, out_hbm,
                       row_v, lab_v, xs_v, acc_v, rsem, lsem, xsem):
    wid = lax.axis_index("s") * _NC + lax.axis_index("c")
    f0 = (half * _NW + wid) * _FPW

    lab_cp = pltpu.async_copy(label_hbm, lab_v, lsem)
    row_cp = pltpu.async_copy(center_t_hbm.at[f0], row_v, rsem)
    xs_cps = [pltpu.async_copy(
        xs_t_hbm.at[f0, pl.ds(0, _CHUNK)], xs_v.at[0], xsem), None]
    lab_cp.wait()

    zeros = jnp.zeros((_L,), jnp.float32)
    accs = (zeros,) * _GRP
    for fi in range(_FPW):
        f = f0 + fi
        row_cp.wait()
        for c in range(4):
            nf, nc = (f, c + 1) if c < 3 else (f + 1, 0)
            buf, nbuf = c % 2, 1 - (c % 2)
            if not (fi == _FPW - 1 and c == 3):
                xs_cps[nbuf] = pltpu.async_copy(
                    xs_t_hbm.at[nf, pl.ds(nc * _CHUNK, _CHUNK)],
                    xs_v.at[nbuf], xsem)
            xs_cps[buf].wait()
            lab_base = c * _CHUNK

            def body(i, accs, lab_base=lab_base, buf=buf):
                out = []
                for g in range(_GRP):
                    o = i * (_L * _GRP) + g * _L
                    idx = lab_v[pl.ds(lab_base + o, _L)]
                    gathered = plsc.load_gather(row_v, [idx])
                    d = xs_v[buf, pl.ds(o, _L)] - gathered
                    out.append(accs[g] + d * d)
                return tuple(out)

            accs = lax.fori_loop(0, _ITERS, body, accs)
        if fi + 1 < _FPW:
            row_cp = pltpu.async_copy(center_t_hbm.at[f + 1], row_v, rsem)

    acc_v[...] = (accs[0] + accs[1]) + (accs[2] + accs[3])
    pltpu.sync_copy(acc_v, out_hbm.at[pl.ds(wid * _L, _L)])
  return _center_partials


def _tc_reduce_body(p_ref, o_ref):
    o_ref[...] = (jnp.sum(p_ref[...]) * (2.0 / BATCH_N))[None, None]


_center_partials_0 = _make_center_partials(0)
_center_partials_1 = _make_center_partials(1)


def kernel(xs, label, center):
    xs_t = xs.T
    lab = label.astype(jnp.int32)
    cen_t = center.T
    p0 = _center_partials_0(xs_t, lab, cen_t)
    p1 = _center_partials_1(xs_t, lab, cen_t)
    loss = pl.pallas_call(
        _tc_reduce_body,
        out_shape=jax.ShapeDtypeStruct((1, 1), jnp.float32),
    )(jnp.concatenate([p0, p1]))
    return loss.reshape((1,))


# speedup vs baseline: 1.3234x; 1.3234x over previous
import functools

import jax
import jax.numpy as jnp
from jax import lax
from jax.experimental import pallas as pl
from jax.experimental.pallas import tpu as pltpu
from jax.experimental.pallas import tpu_sc as plsc

CLS = 100000
FEAT = 64
BATCH_N = 16384

_NC = 2
_NS = 16
_NW = _NC * _NS
_FPW = FEAT // _NW
_L = 16
_CHUNK = BATCH_N // 4
_GRP = 4
_ITERS = _CHUNK // (_L * _GRP)


@functools.partial(
    pl.kernel,
    out_type=jax.ShapeDtypeStruct((_NW * _L,), jnp.float32),
    mesh=plsc.VectorSubcoreMesh(
        core_axis_name="c", subcore_axis_name="s",
        num_cores=_NC, num_subcores=_NS,
    ),
    scratch_types=[
        pltpu.VMEM((CLS,), jnp.float32),
        pltpu.VMEM((BATCH_N,), jnp.int32),
        pltpu.VMEM((2, _CHUNK), jnp.float32),
        pltpu.VMEM((_L,), jnp.float32),
        pltpu.VMEM_SHARED((BATCH_N,), jnp.int32),
        pltpu.SemaphoreType.DMA,
        pltpu.SemaphoreType.DMA,
        pltpu.SemaphoreType.DMA,
    ],
    compiler_params=pltpu.CompilerParams(needs_layout_passes=False),
)
def _center_partials(xs_t_hbm, label_hbm, center_t_hbm, out_hbm,
                     row_v, lab_v, xs_v, acc_v, lab_sh, rsem, lsem, xsem):
    sid = lax.axis_index("s")
    wid = sid * _NC + lax.axis_index("c")
    f0 = wid * _FPW
    seg = BATCH_N // _NS

    lab_cp = pltpu.async_copy(
        label_hbm.at[pl.ds(sid * seg, seg)],
        lab_sh.at[pl.ds(sid * seg, seg)], lsem)
    row_cp = pltpu.async_copy(center_t_hbm.at[f0], row_v, rsem)
    xs_cps = [pltpu.async_copy(
        xs_t_hbm.at[f0, pl.ds(0, _CHUNK)], xs_v.at[0], xsem), None]
    lab_cp.wait()
    plsc.subcore_barrier()
    pltpu.sync_copy(lab_sh, lab_v)

    zeros = jnp.zeros((_L,), jnp.float32)
    accs = (zeros,) * _GRP
    for fi in range(_FPW):
        f = f0 + fi
        row_cp.wait()
        for c in range(4):
            nf, nc = (f, c + 1) if c < 3 else (f + 1, 0)
            buf, nbuf = c % 2, 1 - (c % 2)
            if not (fi == _FPW - 1 and c == 3):
                xs_cps[nbuf] = pltpu.async_copy(
                    xs_t_hbm.at[nf, pl.ds(nc * _CHUNK, _CHUNK)],
                    xs_v.at[nbuf], xsem)
            xs_cps[buf].wait()
            lab_base = c * _CHUNK

            def body(i, accs, lab_base=lab_base, buf=buf):
                out = []
                for g in range(_GRP):
                    o = i * (_L * _GRP) + g * _L
                    idx = lab_v[pl.ds(lab_base + o, _L)]
                    gathered = plsc.load_gather(row_v, [idx])
                    d = xs_v[buf, pl.ds(o, _L)] - gathered
                    out.append(accs[g] + d * d)
                return tuple(out)

            accs = lax.fori_loop(0, _ITERS, body, accs)
        if fi + 1 < _FPW:
            row_cp = pltpu.async_copy(center_t_hbm.at[f + 1], row_v, rsem)

    acc_v[...] = (accs[0] + accs[1]) + (accs[2] + accs[3])
    pltpu.sync_copy(acc_v, out_hbm.at[pl.ds(wid * _L, _L)])


def _tc_reduce_body(p_ref, o_ref):
    o_ref[...] = (jnp.sum(p_ref[...]) * (2.0 / BATCH_N))[None, None]


def kernel(xs, label, center):
    partials = _center_partials(xs.T, label.astype(jnp.int32), center.T)
    loss = pl.pallas_call(
        _tc_reduce_body,
        out_shape=jax.ShapeDtypeStruct((1, 1), jnp.float32),
    )(partials)
    return loss.reshape((1,))
